# baseline (speedup 1.0000x reference)
import functools

import jax
import jax.numpy as jnp
from jax import lax
from jax.experimental import pallas as pl
from jax.experimental.pallas import tpu as pltpu
from jax.experimental.pallas import tpu_sc as plsc

_NC = 2
_NS = 16
_NW = _NC * _NS
_K = 16


def _gather_sum_body(table_hbm, nbr_hbm, out_hbm, idx_v, buf0, buf1,
                     acc0, acc1, sem0, sem1, osem0, osem1,
                     *, d, cw, rw, ips):
    gpc = ips // _K
    wid = lax.axis_index("s") * _NC + lax.axis_index("c")
    bufs = (buf0, buf1)
    accs = (acc0, acc1)
    sems = (sem0, sem1)
    osems = (osem0, osem1)
    pltpu.sync_copy(nbr_hbm.at[pl.ds(wid * cw, cw), :], idx_v)

    def reduce_chunk(b, c):
        buf, acc = bufs[b], accs[b]
        for g in range(gpc):
            base = g * _K
            for col in range(d // 16):
                cs = pl.ds(col * 16, 16)
                v = [buf[base + r, cs] for r in range(_K)]
                while len(v) > 1:
                    v = [v[2 * j] + v[2 * j + 1] for j in range(len(v) // 2)]
                acc[g, cs] = v[0]
        pltpu.async_copy(acc, out_hbm.at[pl.ds(wid * rw + c * gpc, gpc), :],
                         osems[b])

    def wait_out(b):
        pltpu.make_async_copy(accs[b], out_hbm.at[pl.ds(wid * rw, gpc), :],
                              osems[b]).wait()

    def gather(c, b):
        pltpu.async_copy(table_hbm.at[idx_v.at[c]], bufs[b], sems[b])

    def wait_gather(c, b):
        pltpu.make_async_copy(table_hbm.at[idx_v.at[c]], bufs[b],
                              sems[b]).wait()

    gather(0, 0)

    def pair(i, _):
        c0 = i * 2
        wait_gather(c0, 0)
        gather(c0 + 1, 1)

        @pl.when(i > 0)
        def _():
            wait_out(0)

        reduce_chunk(0, c0)
        wait_gather(c0 + 1, 1)

        @pl.when(c0 + 2 < cw)
        def _():
            gather(c0 + 2, 0)

        @pl.when(i > 0)
        def _():
            wait_out(1)

        reduce_chunk(1, c0 + 1)
        return 0

    lax.fori_loop(0, cw // 2, pair, 0)
    wait_out(0)
    wait_out(1)


def _gather_sum(table, nbr_flat, n_pad, d):
    ips = 128 if d <= 256 else 64
    g_per_chunk = ips // _K
    rw = n_pad // _NW
    cw = rw // g_per_chunk
    nbr_blocks = nbr_flat.reshape(-1, ips)
    mesh = plsc.VectorSubcoreMesh(core_axis_name="c", subcore_axis_name="s")
    body = functools.partial(_gather_sum_body, d=d, cw=cw, rw=rw, ips=ips)
    return pl.kernel(
        body,
        mesh=mesh,
        out_type=jax.ShapeDtypeStruct((n_pad, d), jnp.float32),
        scratch_types=[
            pltpu.VMEM((cw, ips), jnp.int32),
            pltpu.VMEM((ips, d), jnp.float32),
            pltpu.VMEM((ips, d), jnp.float32),
            pltpu.VMEM((g_per_chunk, d), jnp.float32),
            pltpu.VMEM((g_per_chunk, d), jnp.float32),
            pltpu.SemaphoreType.DMA,
            pltpu.SemaphoreType.DMA,
            pltpu.SemaphoreType.DMA,
            pltpu.SemaphoreType.DMA,
        ],
        name=f"sage_gather_sum_d{d}",
    )(table, nbr_blocks)


def _combine_body(feat_ref, agg_ref, w_ref, out_ref, *, d_in, inv_k):
    x = feat_ref[...]
    a = agg_ref[...]
    ws = w_ref[:, :d_in]
    wn = w_ref[:, d_in:]
    y = lax.dot_general(x, ws, (((1,), (1,)), ((), ())),
                        preferred_element_type=jnp.float32)
    y = y + inv_k * lax.dot_general(a, wn, (((1,), (1,)), ((), ())),
                                    preferred_element_type=jnp.float32)
    out_ref[...] = jnp.maximum(y, 0.0)


def _combine(feat, agg_sum, w, bm):
    n, d_in = feat.shape
    d_out = w.shape[0]
    grid = n // bm
    body = functools.partial(_combine_body, d_in=d_in, inv_k=1.0 / _K)
    return pl.pallas_call(
        body,
        grid=(grid,),
        in_specs=[
            pl.BlockSpec((bm, d_in), lambda i: (i, 0)),
            pl.BlockSpec((bm, d_in), lambda i: (i, 0)),
            pl.BlockSpec((d_out, 2 * d_in), lambda i: (0, 0)),
        ],
        out_specs=pl.BlockSpec((bm, d_out), lambda i: (i, 0)),
        out_shape=jax.ShapeDtypeStruct((n, d_out), jnp.float32),
        name=f"sage_combine_{d_in}",
    )(feat, agg_sum, w)


def kernel(nodes, feat_data, neighbors, W0, W1):
    del nodes
    n, d_in = feat_data.shape
    d_out = W0.shape[0]

    n_pad = ((n + _NW * 8 - 1) // (_NW * 8)) * (_NW * 8)
    pad = n_pad - n
    nbr = neighbors.astype(jnp.int32).reshape(-1)
    if pad:
        dummy = (jnp.arange(pad * _K, dtype=jnp.int32) * 97) % n
        nbr = jnp.concatenate([nbr, dummy])

    bm = 400 if n % 400 == 0 else max(
        b for b in (512, 256, 200, 128, 100, 80, 50, 40, 25, 20, 16, 10, 8, 5, 4, 2, 1)
        if n % b == 0)

    agg0 = _gather_sum(feat_data, nbr, n_pad, d_in)
    h1 = _combine(feat_data, agg0, W0, bm)
    agg1 = _gather_sum(h1, nbr, n_pad, d_out)
    return _combine(h1, agg1, W1, bm)


# speedup vs baseline: 2.3185x; 2.3185x over previous
import functools

import jax
import jax.numpy as jnp
from jax import lax
from jax.experimental import pallas as pl
from jax.experimental.pallas import tpu as pltpu
from jax.experimental.pallas import tpu_sc as plsc

_NC = 2
_NS = 16
_NW = _NC * _NS
_K = 16


def _gather_sum_body(table_hbm, nbr_hbm, out_hbm, idx_v, buf0, buf1,
                     acc0, acc1, sem0, sem1, osem0, osem1,
                     *, d, cw, rw, ips):
    gpc = ips // _K
    wid = lax.axis_index("s") * _NC + lax.axis_index("c")
    bufs = (buf0, buf1)
    accs = (acc0, acc1)
    sems = (sem0, sem1)
    osems = (osem0, osem1)
    pltpu.sync_copy(nbr_hbm.at[pl.ds(wid * cw, cw), :], idx_v)

    def reduce_chunk(b, c):
        buf, acc = bufs[b], accs[b]

        def per_node(g, _):
            base = g * _K
            for col in range(d // 16):
                cs = pl.ds(col * 16, 16)
                v = [buf[base + r, cs] for r in range(_K)]
                while len(v) > 1:
                    v = [v[2 * j] + v[2 * j + 1] for j in range(len(v) // 2)]
                acc[g, cs] = v[0]
            return 0

        lax.fori_loop(0, gpc, per_node, 0)
        pltpu.async_copy(acc, out_hbm.at[pl.ds(wid * rw + c * gpc, gpc), :],
                         osems[b])

    def wait_out(b):
        pltpu.make_async_copy(accs[b], out_hbm.at[pl.ds(wid * rw, gpc), :],
                              osems[b]).wait()

    def gather(c, b):
        pltpu.async_copy(table_hbm.at[idx_v.at[c]], bufs[b], sems[b])

    def wait_gather(c, b):
        pltpu.make_async_copy(table_hbm.at[idx_v.at[c]], bufs[b],
                              sems[b]).wait()

    gather(0, 0)

    def pair(i, _):
        c0 = i * 2
        wait_gather(c0, 0)
        gather(c0 + 1, 1)

        @pl.when(i > 0)
        def _():
            wait_out(0)

        reduce_chunk(0, c0)
        wait_gather(c0 + 1, 1)

        @pl.when(c0 + 2 < cw)
        def _():
            gather(c0 + 2, 0)

        @pl.when(i > 0)
        def _():
            wait_out(1)

        reduce_chunk(1, c0 + 1)
        return 0

    lax.fori_loop(0, cw // 2, pair, 0)
    wait_out(0)
    wait_out(1)


def _gather_sum(table, nbr_flat, n_pad, d):
    ips = 128 if d <= 256 else 64
    g_per_chunk = ips // _K
    rw = n_pad // _NW
    cw = rw // g_per_chunk
    nbr_blocks = nbr_flat.reshape(-1, ips)
    mesh = plsc.VectorSubcoreMesh(core_axis_name="c", subcore_axis_name="s")
    body = functools.partial(_gather_sum_body, d=d, cw=cw, rw=rw, ips=ips)
    return pl.kernel(
        body,
        mesh=mesh,
        out_type=jax.ShapeDtypeStruct((n_pad, d), jnp.float32),
        scratch_types=[
            pltpu.VMEM((cw, ips), jnp.int32),
            pltpu.VMEM((ips, d), jnp.float32),
            pltpu.VMEM((ips, d), jnp.float32),
            pltpu.VMEM((g_per_chunk, d), jnp.float32),
            pltpu.VMEM((g_per_chunk, d), jnp.float32),
            pltpu.SemaphoreType.DMA,
            pltpu.SemaphoreType.DMA,
            pltpu.SemaphoreType.DMA,
            pltpu.SemaphoreType.DMA,
        ],
        name=f"sage_gather_sum_d{d}",
    )(table, nbr_blocks)


def _combine_body(feat_ref, agg_ref, w_ref, out_ref, *, d_in, inv_k):
    x = feat_ref[...]
    a = agg_ref[...]
    ws = w_ref[:, :d_in]
    wn = w_ref[:, d_in:]
    y = lax.dot_general(x, ws, (((1,), (1,)), ((), ())),
                        preferred_element_type=jnp.float32)
    y = y + inv_k * lax.dot_general(a, wn, (((1,), (1,)), ((), ())),
                                    preferred_element_type=jnp.float32)
    out_ref[...] = jnp.maximum(y, 0.0)


def _combine(feat, agg_sum, w, bm):
    n, d_in = feat.shape
    d_out = w.shape[0]
    grid = n // bm
    body = functools.partial(_combine_body, d_in=d_in, inv_k=1.0 / _K)
    return pl.pallas_call(
        body,
        grid=(grid,),
        in_specs=[
            pl.BlockSpec((bm, d_in), lambda i: (i, 0)),
            pl.BlockSpec((bm, d_in), lambda i: (i, 0)),
            pl.BlockSpec((d_out, 2 * d_in), lambda i: (0, 0)),
        ],
        out_specs=pl.BlockSpec((bm, d_out), lambda i: (i, 0)),
        out_shape=jax.ShapeDtypeStruct((n, d_out), jnp.float32),
        name=f"sage_combine_{d_in}",
    )(feat, agg_sum, w)


def kernel(nodes, feat_data, neighbors, W0, W1):
    del nodes
    n, d_in = feat_data.shape
    d_out = W0.shape[0]

    n_pad = ((n + _NW * 8 - 1) // (_NW * 8)) * (_NW * 8)
    pad = n_pad - n
    nbr = neighbors.astype(jnp.int32).reshape(-1)
    if pad:
        dummy = (jnp.arange(pad * _K, dtype=jnp.int32) * 97) % n
        nbr = jnp.concatenate([nbr, dummy])

    bm = 400 if n % 400 == 0 else max(
        b for b in (512, 256, 200, 128, 100, 80, 50, 40, 25, 20, 16, 10, 8, 5, 4, 2, 1)
        if n % b == 0)

    agg0 = _gather_sum(feat_data, nbr, n_pad, d_in)
    h1 = _combine(feat_data, agg0, W0, bm)
    agg1 = _gather_sum(h1, nbr, n_pad, d_out)
    return _combine(h1, agg1, W1, bm)
